# baseline (device time: 202176 ns/iter reference)
import jax
import jax.numpy as jnp
from jax import lax
from jax.experimental import pallas as pl
from jax.experimental.pallas import tpu as pltpu

N_DEV = 4
TAPS = 4
HALO = 8


def _silu(v):
    return v * jax.nn.sigmoid(v)


def _fused_body(
    x_ref, k_ref, w_ref, o_ref,
    comm_cw, comm_ccw, xs, xb, stage_cw, stage_ccw,
    send_cw, recv_cw, send_ccw, recv_ccw, x_sem_t, x_sem_b, st_cw, st_ccw,
):
    d = lax.axis_index("i")
    left = (d - 1) % N_DEV
    right = (d + 1) % N_DEV
    B, S, C = x_ref.shape
    H = S // 2
    f32 = jnp.float32
    bf16 = jnp.bfloat16
    w_bf = w_ref[...].astype(bf16)

    def load_top(b):
        cp = pltpu.make_async_copy(x_ref.at[b, pl.ds(0, H)], xs, x_sem_t)
        cp.start()
        return cp

    def load_bot(b):
        cp = pltpu.make_async_copy(
            x_ref.at[b, pl.ds(H - HALO, H + HALO)], xb, x_sem_b)
        cp.start()
        return cp

    def product_top():
        xv = xs[...]
        conv = xv * k_ref[TAPS - 1]
        for t in range(TAPS - 1):
            m = TAPS - 1 - t
            conv += jnp.concatenate(
                [jnp.zeros((m, C), f32), xv[: H - m]], axis=0
            ) * k_ref[t]
        return jnp.dot(
            _silu(conv).astype(bf16), w_bf, preferred_element_type=f32
        )

    def product_bot():
        xv = xb[...]
        conv = xv[HALO: HALO + H] * k_ref[TAPS - 1]
        for t in range(TAPS - 1):
            m = TAPS - 1 - t
            conv += xv[HALO - m: HALO - m + H] * k_ref[t]
        return jnp.dot(
            _silu(conv).astype(bf16), w_bf, preferred_element_type=f32
        )

    cpT = load_top(d)
    cpB = load_bot(d)
    cpT.wait()
    comm_cw[0] = product_top().astype(bf16)
    cpB.wait()
    comm_ccw[0] = product_bot().astype(bf16)

    barrier = pltpu.get_barrier_semaphore()
    for nbr in (left, right):
        pl.semaphore_signal(
            barrier, inc=1, device_id=(nbr,),
            device_id_type=pl.DeviceIdType.MESH,
        )
    pl.semaphore_wait(barrier, 2)

    for h in range(N_DEV - 1):
        rs = (h + 1) % 2
        rdma_cw = pltpu.make_async_remote_copy(
            src_ref=comm_cw.at[h % 2],
            dst_ref=comm_cw.at[rs],
            send_sem=send_cw.at[h],
            recv_sem=recv_cw.at[h],
            device_id=(right,),
            device_id_type=pl.DeviceIdType.MESH,
        )
        rdma_ccw = pltpu.make_async_remote_copy(
            src_ref=comm_ccw.at[h % 2],
            dst_ref=comm_ccw.at[rs],
            send_sem=send_ccw.at[h],
            recv_sem=recv_ccw.at[h],
            device_id=(left,),
            device_id_type=pl.DeviceIdType.MESH,
        )
        rdma_cw.start()
        cpT = load_top((d - h - 1) % N_DEV)
        rdma_ccw.start()
        cpB = load_bot((d + h + 1) % N_DEV)
        cpT.wait()
        add_cw = product_top()
        rdma_cw.wait()
        comm_cw[rs] = (comm_cw[rs].astype(f32) + add_cw).astype(bf16)
        cpB.wait()
        add_ccw = product_bot()
        rdma_ccw.wait()
        comm_ccw[rs] = (comm_ccw[rs].astype(f32) + add_ccw).astype(bf16)

    stores_cw = []
    stores_ccw = []
    pending = (1, (d + 1) % N_DEV, (d - 1) % N_DEV)
    for j in range(N_DEV - 1):
        h = N_DEV - 1 + j
        rs = (h + 1) % 2
        rdma_cw = pltpu.make_async_remote_copy(
            src_ref=comm_cw.at[h % 2],
            dst_ref=comm_cw.at[rs],
            send_sem=send_cw.at[h],
            recv_sem=recv_cw.at[h],
            device_id=(right,),
            device_id_type=pl.DeviceIdType.MESH,
        )
        rdma_ccw = pltpu.make_async_remote_copy(
            src_ref=comm_ccw.at[h % 2],
            dst_ref=comm_ccw.at[rs],
            send_sem=send_ccw.at[h],
            recv_sem=recv_ccw.at[h],
            device_id=(left,),
            device_id_type=pl.DeviceIdType.MESH,
        )
        rdma_cw.start()
        rdma_ccw.start()
        slot, c_cw, c_ccw = pending
        s = j
        if s >= 2:
            stores_cw[s - 2].wait()
            stores_ccw[s - 2].wait()
        stage_cw[s % 2] = comm_cw[slot].astype(f32)
        st = pltpu.make_async_copy(
            stage_cw.at[s % 2], o_ref.at[c_cw, pl.ds(0, H)], st_cw.at[s])
        st.start()
        stores_cw.append(st)
        stage_ccw[s % 2] = comm_ccw[slot].astype(f32)
        st = pltpu.make_async_copy(
            stage_ccw.at[s % 2], o_ref.at[c_ccw, pl.ds(H, H)], st_ccw.at[s])
        st.start()
        stores_ccw.append(st)
        rdma_cw.wait()
        rdma_ccw.wait()
        pending = (rs, (d - j) % N_DEV, (d + j) % N_DEV)

    slot, c_cw, c_ccw = pending
    s = N_DEV - 1
    stores_cw[s - 2].wait()
    stores_ccw[s - 2].wait()
    stage_cw[s % 2] = comm_cw[slot].astype(f32)
    st = pltpu.make_async_copy(
        stage_cw.at[s % 2], o_ref.at[c_cw, pl.ds(0, H)], st_cw.at[s])
    st.start()
    stores_cw.append(st)
    stage_ccw[s % 2] = comm_ccw[slot].astype(f32)
    st = pltpu.make_async_copy(
        stage_ccw.at[s % 2], o_ref.at[c_ccw, pl.ds(H, H)], st_ccw.at[s])
    st.start()
    stores_ccw.append(st)

    for st in stores_cw[-2:] + stores_ccw[-2:]:
        st.wait()


def kernel(x, k, Wp):
    B, S, C = x.shape
    Co = Wp.shape[1]
    H = S // 2
    n_hops = 2 * (N_DEV - 1)
    return pl.pallas_call(
        _fused_body,
        in_specs=[
            pl.BlockSpec(memory_space=pl.ANY),
            pl.BlockSpec(memory_space=pltpu.VMEM),
            pl.BlockSpec(memory_space=pltpu.VMEM),
        ],
        out_specs=pl.BlockSpec(memory_space=pl.ANY),
        out_shape=jax.ShapeDtypeStruct((B, S, Co), jnp.float32),
        scratch_shapes=[
            pltpu.VMEM((2, H, Co), jnp.bfloat16),
            pltpu.VMEM((2, H, Co), jnp.bfloat16),
            pltpu.VMEM((H, C), jnp.float32),
            pltpu.VMEM((H + HALO, C), jnp.float32),
            pltpu.VMEM((2, H, Co), jnp.float32),
            pltpu.VMEM((2, H, Co), jnp.float32),
            pltpu.SemaphoreType.DMA((n_hops,)),
            pltpu.SemaphoreType.DMA((n_hops,)),
            pltpu.SemaphoreType.DMA((n_hops,)),
            pltpu.SemaphoreType.DMA((n_hops,)),
            pltpu.SemaphoreType.DMA,
            pltpu.SemaphoreType.DMA,
            pltpu.SemaphoreType.DMA((4,)),
            pltpu.SemaphoreType.DMA((4,)),
        ],
        compiler_params=pltpu.CompilerParams(
            collective_id=0, vmem_limit_bytes=110 * 1024 * 1024
        ),
    )(x, k, Wp)


# device time: 189421 ns/iter; 1.0673x vs baseline; 1.0673x over previous
import jax
import jax.numpy as jnp
from jax import lax
from jax.experimental import pallas as pl
from jax.experimental.pallas import tpu as pltpu

N_DEV = 4
TAPS = 4
HALO = 8


def _silu(v):
    return v * jax.nn.sigmoid(v)


def _fused_body(
    x_ref, k_ref, w_ref, o_ref,
    comm_cw, comm_ccw, xs, xb, stage_cw, stage_ccw,
    send_cw, recv_cw, send_ccw, recv_ccw, x_sem_t, x_sem_b, st_cw, st_ccw,
):
    d = lax.axis_index("i")
    left = (d - 1) % N_DEV
    right = (d + 1) % N_DEV
    B, S, C = x_ref.shape
    H = S // 2
    f32 = jnp.float32
    bf16 = jnp.bfloat16
    w_bf = w_ref[...].astype(bf16)

    def load_top(b):
        cp = pltpu.make_async_copy(x_ref.at[b, pl.ds(0, H)], xs, x_sem_t)
        cp.start()
        return cp

    def load_bot(b):
        cp = pltpu.make_async_copy(
            x_ref.at[b, pl.ds(H - HALO, H + HALO)], xb, x_sem_b)
        cp.start()
        return cp

    def product_top():
        xv = xs[...]
        conv = xv * k_ref[TAPS - 1]
        for t in range(TAPS - 1):
            m = TAPS - 1 - t
            conv += jnp.concatenate(
                [jnp.zeros((m, C), f32), xv[: H - m]], axis=0
            ) * k_ref[t]
        return jnp.dot(
            _silu(conv).astype(bf16), w_bf, preferred_element_type=f32
        )

    def product_bot():
        xv = xb[...]
        conv = xv[HALO: HALO + H] * k_ref[TAPS - 1]
        for t in range(TAPS - 1):
            m = TAPS - 1 - t
            conv += xv[HALO - m: HALO - m + H] * k_ref[t]
        return jnp.dot(
            _silu(conv).astype(bf16), w_bf, preferred_element_type=f32
        )

    cpT = load_top(d)
    cpB = load_bot(d)
    cpT.wait()
    comm_cw[0] = product_top().astype(bf16)
    cpB.wait()
    comm_ccw[0] = product_bot().astype(bf16)

    barrier = pltpu.get_barrier_semaphore()
    for nbr in (left, right):
        pl.semaphore_signal(
            barrier, inc=1, device_id=(nbr,),
            device_id_type=pl.DeviceIdType.MESH,
        )
    pl.semaphore_wait(barrier, 2)

    for h in range(N_DEV - 1):
        rs = (h + 1) % 2
        rdma_cw = pltpu.make_async_remote_copy(
            src_ref=comm_cw.at[h % 2],
            dst_ref=comm_cw.at[rs],
            send_sem=send_cw.at[h],
            recv_sem=recv_cw.at[h],
            device_id=(right,),
            device_id_type=pl.DeviceIdType.MESH,
        )
        rdma_ccw = pltpu.make_async_remote_copy(
            src_ref=comm_ccw.at[h % 2],
            dst_ref=comm_ccw.at[rs],
            send_sem=send_ccw.at[h],
            recv_sem=recv_ccw.at[h],
            device_id=(left,),
            device_id_type=pl.DeviceIdType.MESH,
        )
        rdma_cw.start()
        rdma_ccw.start()
        cpT = load_top((d - h - 1) % N_DEV)
        cpB = load_bot((d + h + 1) % N_DEV)
        cpT.wait()
        add_cw = product_top()
        cpB.wait()
        add_ccw = product_bot()
        rdma_cw.wait()
        rdma_ccw.wait()
        comm_cw[rs] = (comm_cw[rs].astype(f32) + add_cw).astype(bf16)
        comm_ccw[rs] = (comm_ccw[rs].astype(f32) + add_ccw).astype(bf16)

    stores_cw = []
    stores_ccw = []
    pending = (1, (d + 1) % N_DEV, (d - 1) % N_DEV)
    for j in range(N_DEV - 1):
        h = N_DEV - 1 + j
        rs = (h + 1) % 2
        rdma_cw = pltpu.make_async_remote_copy(
            src_ref=comm_cw.at[h % 2],
            dst_ref=comm_cw.at[rs],
            send_sem=send_cw.at[h],
            recv_sem=recv_cw.at[h],
            device_id=(right,),
            device_id_type=pl.DeviceIdType.MESH,
        )
        rdma_ccw = pltpu.make_async_remote_copy(
            src_ref=comm_ccw.at[h % 2],
            dst_ref=comm_ccw.at[rs],
            send_sem=send_ccw.at[h],
            recv_sem=recv_ccw.at[h],
            device_id=(left,),
            device_id_type=pl.DeviceIdType.MESH,
        )
        rdma_cw.start()
        rdma_ccw.start()
        slot, c_cw, c_ccw = pending
        s = j
        if s >= 2:
            stores_cw[s - 2].wait()
            stores_ccw[s - 2].wait()
        stage_cw[s % 2] = comm_cw[slot].astype(f32)
        st = pltpu.make_async_copy(
            stage_cw.at[s % 2], o_ref.at[c_cw, pl.ds(0, H)], st_cw.at[s])
        st.start()
        stores_cw.append(st)
        stage_ccw[s % 2] = comm_ccw[slot].astype(f32)
        st = pltpu.make_async_copy(
            stage_ccw.at[s % 2], o_ref.at[c_ccw, pl.ds(H, H)], st_ccw.at[s])
        st.start()
        stores_ccw.append(st)
        rdma_cw.wait()
        rdma_ccw.wait()
        pending = (rs, (d - j) % N_DEV, (d + j) % N_DEV)

    slot, c_cw, c_ccw = pending
    s = N_DEV - 1
    stores_cw[s - 2].wait()
    stores_ccw[s - 2].wait()
    stage_cw[s % 2] = comm_cw[slot].astype(f32)
    st = pltpu.make_async_copy(
        stage_cw.at[s % 2], o_ref.at[c_cw, pl.ds(0, H)], st_cw.at[s])
    st.start()
    stores_cw.append(st)
    stage_ccw[s % 2] = comm_ccw[slot].astype(f32)
    st = pltpu.make_async_copy(
        stage_ccw.at[s % 2], o_ref.at[c_ccw, pl.ds(H, H)], st_ccw.at[s])
    st.start()
    stores_ccw.append(st)

    for st in stores_cw[-2:] + stores_ccw[-2:]:
        st.wait()


def kernel(x, k, Wp):
    B, S, C = x.shape
    Co = Wp.shape[1]
    H = S // 2
    n_hops = 2 * (N_DEV - 1)
    return pl.pallas_call(
        _fused_body,
        in_specs=[
            pl.BlockSpec(memory_space=pl.ANY),
            pl.BlockSpec(memory_space=pltpu.VMEM),
            pl.BlockSpec(memory_space=pltpu.VMEM),
        ],
        out_specs=pl.BlockSpec(memory_space=pl.ANY),
        out_shape=jax.ShapeDtypeStruct((B, S, Co), jnp.float32),
        scratch_shapes=[
            pltpu.VMEM((2, H, Co), jnp.bfloat16),
            pltpu.VMEM((2, H, Co), jnp.bfloat16),
            pltpu.VMEM((H, C), jnp.float32),
            pltpu.VMEM((H + HALO, C), jnp.float32),
            pltpu.VMEM((2, H, Co), jnp.float32),
            pltpu.VMEM((2, H, Co), jnp.float32),
            pltpu.SemaphoreType.DMA((n_hops,)),
            pltpu.SemaphoreType.DMA((n_hops,)),
            pltpu.SemaphoreType.DMA((n_hops,)),
            pltpu.SemaphoreType.DMA((n_hops,)),
            pltpu.SemaphoreType.DMA,
            pltpu.SemaphoreType.DMA,
            pltpu.SemaphoreType.DMA((4,)),
            pltpu.SemaphoreType.DMA((4,)),
        ],
        compiler_params=pltpu.CompilerParams(
            collective_id=0, vmem_limit_bytes=110 * 1024 * 1024
        ),
    )(x, k, Wp)


# device time: 176053 ns/iter; 1.1484x vs baseline; 1.0759x over previous
import jax
import jax.numpy as jnp
from jax import lax
from jax.experimental import pallas as pl
from jax.experimental.pallas import tpu as pltpu

N_DEV = 4
TAPS = 4
HALO = 8
Q = 2
CW, CCW = 0, 1


def _silu(v):
    return v * jax.nn.sigmoid(v)


def _fused_body(
    x_ref, k_ref, w_ref, o_ref,
    comm_cw, comm_ccw, xs, xb, stage_cw, stage_ccw,
    send_cw, recv_cw, send_ccw, recv_ccw, x_sem_t, x_sem_b, st_cw, st_ccw,
):
    d = lax.axis_index("i")
    left = (d - 1) % N_DEV
    right = (d + 1) % N_DEV
    B, S, C = x_ref.shape
    Co = w_ref.shape[1]
    H = S // 2
    Hq = H // Q
    f32 = jnp.float32
    bf16 = jnp.bfloat16
    w_bf = w_ref[...].astype(bf16)

    def R(q):
        return pl.ds(q * Hq, Hq)

    def load_top(b):
        cp = pltpu.make_async_copy(x_ref.at[b, pl.ds(0, H)], xs, x_sem_t)
        cp.start()
        return cp

    def load_bot(b):
        cp = pltpu.make_async_copy(
            x_ref.at[b, pl.ds(H - HALO, H + HALO)], xb, x_sem_b)
        cp.start()
        return cp

    def product_sub(xv, base, q, edge):
        conv = xv[base + q * Hq: base + (q + 1) * Hq] * k_ref[TAPS - 1]
        for t in range(TAPS - 1):
            m = TAPS - 1 - t
            start = base + q * Hq - m
            if edge and start < 0:
                sl = jnp.concatenate(
                    [jnp.zeros((m, C), f32), xv[: Hq - m]], axis=0)
            else:
                sl = xv[start: start + Hq]
            conv += sl * k_ref[t]
        return jnp.dot(
            _silu(conv).astype(bf16), w_bf, preferred_element_type=f32)

    sends = {}

    def rdma(dir_, h, q):
        comm = comm_cw if dir_ == CW else comm_ccw
        snd = send_cw if dir_ == CW else send_ccw
        rcv = recv_cw if dir_ == CW else recv_ccw
        tgt = right if dir_ == CW else left
        return pltpu.make_async_remote_copy(
            src_ref=comm.at[h % 2, R(q)],
            dst_ref=comm.at[(h + 1) % 2, R(q)],
            send_sem=snd.at[h, q],
            recv_sem=rcv.at[h, q],
            device_id=(tgt,),
            device_id_type=pl.DeviceIdType.MESH,
        )

    def start_send(dir_, h, q):
        desc = rdma(dir_, h, q)
        desc.start()
        sends[(dir_, h, q)] = desc

    barrier = pltpu.get_barrier_semaphore()
    for nbr in (left, right):
        pl.semaphore_signal(
            barrier, inc=1, device_id=(nbr,),
            device_id_type=pl.DeviceIdType.MESH,
        )
    pl.semaphore_wait(barrier, 2)

    cpT = load_top(d)
    cpB = load_bot(d)
    cpT.wait()
    xvT = xs[...]
    comm_cw[0, R(0)] = product_sub(xvT, 0, 0, True).astype(bf16)
    start_send(CW, 0, 0)
    cpB.wait()
    xvB = xb[...]
    comm_ccw[0, R(0)] = product_sub(xvB, HALO, 0, False).astype(bf16)
    start_send(CCW, 0, 0)
    comm_cw[0, R(1)] = product_sub(xvT, 0, 1, True).astype(bf16)
    start_send(CW, 0, 1)
    comm_ccw[0, R(1)] = product_sub(xvB, HALO, 1, False).astype(bf16)
    start_send(CCW, 0, 1)

    for h in range(N_DEV - 1):
        rs_ = (h + 1) % 2
        cpT = load_top((d - h - 1) % N_DEV)
        cpB = load_bot((d + h + 1) % N_DEV)
        cpT.wait()
        xvT = xs[...]
        cpB.wait()
        xvB = xb[...]
        for q in range(Q):
            addT = product_sub(xvT, 0, q, True)
            rdma(CW, h, q).wait_recv()
            if h >= 1:
                sends[(CW, h - 1, q)].wait_send()
            comm_cw[rs_, R(q)] = (
                comm_cw[rs_, R(q)].astype(f32) + addT).astype(bf16)
            start_send(CW, h + 1, q)
            addB = product_sub(xvB, HALO, q, False)
            rdma(CCW, h, q).wait_recv()
            if h >= 1:
                sends[(CCW, h - 1, q)].wait_send()
            comm_ccw[rs_, R(q)] = (
                comm_ccw[rs_, R(q)].astype(f32) + addB).astype(bf16)
            start_send(CCW, h + 1, q)

    stores_cw = []
    stores_ccw = []

    def do_store(pending, s):
        slot, c_cw, c_ccw = pending
        if s >= 2:
            stores_cw[s - 2].wait()
            stores_ccw[s - 2].wait()
        stage_cw[s % 2] = comm_cw[slot].astype(f32)
        st = pltpu.make_async_copy(
            stage_cw.at[s % 2], o_ref.at[c_cw, pl.ds(0, H)], st_cw.at[s])
        st.start()
        stores_cw.append(st)
        stage_ccw[s % 2] = comm_ccw[slot].astype(f32)
        st = pltpu.make_async_copy(
            stage_ccw.at[s % 2], o_ref.at[c_ccw, pl.ds(H, H)], st_ccw.at[s])
        st.start()
        stores_ccw.append(st)

    pending = (1, (d + 1) % N_DEV, (d - 1) % N_DEV)
    for h in (N_DEV - 1, N_DEV):
        do_store(pending, s=h - (N_DEV - 1))
        for q in range(Q):
            rdma(CW, h, q).wait_recv()
            start_send(CW, h + 1, q)
            rdma(CCW, h, q).wait_recv()
            start_send(CCW, h + 1, q)
        j = h - (N_DEV - 1)
        pending = ((h + 1) % 2, (d - j) % N_DEV, (d + j) % N_DEV)

    h = N_DEV + 1
    do_store(pending, s=2)
    for q in range(Q):
        rdma(CW, h, q).wait_recv()
        rdma(CCW, h, q).wait_recv()
    do_store((0, (d - 2) % N_DEV, (d + 2) % N_DEV), s=3)

    for dir_ in (CW, CCW):
        for hh in range(2, 2 * (N_DEV - 1)):
            for q in range(Q):
                sends[(dir_, hh, q)].wait_send()
    for st in stores_cw[-2:] + stores_ccw[-2:]:
        st.wait()


def kernel(x, k, Wp):
    B, S, C = x.shape
    Co = Wp.shape[1]
    H = S // 2
    n_hops = 2 * (N_DEV - 1)
    return pl.pallas_call(
        _fused_body,
        in_specs=[
            pl.BlockSpec(memory_space=pl.ANY),
            pl.BlockSpec(memory_space=pltpu.VMEM),
            pl.BlockSpec(memory_space=pltpu.VMEM),
        ],
        out_specs=pl.BlockSpec(memory_space=pl.ANY),
        out_shape=jax.ShapeDtypeStruct((B, S, Co), jnp.float32),
        scratch_shapes=[
            pltpu.VMEM((2, H, Co), jnp.bfloat16),
            pltpu.VMEM((2, H, Co), jnp.bfloat16),
            pltpu.VMEM((H, C), jnp.float32),
            pltpu.VMEM((H + HALO, C), jnp.float32),
            pltpu.VMEM((2, H, Co), jnp.float32),
            pltpu.VMEM((2, H, Co), jnp.float32),
            pltpu.SemaphoreType.DMA((n_hops, Q)),
            pltpu.SemaphoreType.DMA((n_hops, Q)),
            pltpu.SemaphoreType.DMA((n_hops, Q)),
            pltpu.SemaphoreType.DMA((n_hops, Q)),
            pltpu.SemaphoreType.DMA,
            pltpu.SemaphoreType.DMA,
            pltpu.SemaphoreType.DMA((4,)),
            pltpu.SemaphoreType.DMA((4,)),
        ],
        compiler_params=pltpu.CompilerParams(
            collective_id=0, vmem_limit_bytes=110 * 1024 * 1024
        ),
    )(x, k, Wp)


# device time: 174591 ns/iter; 1.1580x vs baseline; 1.0084x over previous
import jax
import jax.numpy as jnp
from jax import lax
from jax.experimental import pallas as pl
from jax.experimental.pallas import tpu as pltpu

N_DEV = 4
TAPS = 4
HALO = 8
Q = 4
CW, CCW = 0, 1


def _silu(v):
    return v * jax.nn.sigmoid(v)


def _fused_body(
    x_ref, k_ref, w_ref, o_ref,
    comm_cw, comm_ccw, xs, xb, stage_cw, stage_ccw,
    send_cw, recv_cw, send_ccw, recv_ccw, x_sem_t, x_sem_b, st_cw, st_ccw,
):
    d = lax.axis_index("i")
    left = (d - 1) % N_DEV
    right = (d + 1) % N_DEV
    B, S, C = x_ref.shape
    Co = w_ref.shape[1]
    H = S // 2
    Hq = H // Q
    f32 = jnp.float32
    bf16 = jnp.bfloat16
    w_bf = w_ref[...].astype(bf16)

    def R(q):
        return pl.ds(q * Hq, Hq)

    def load_top(b):
        cp = pltpu.make_async_copy(x_ref.at[b, pl.ds(0, H)], xs, x_sem_t)
        cp.start()
        return cp

    def load_bot(b):
        cp = pltpu.make_async_copy(
            x_ref.at[b, pl.ds(H - HALO, H + HALO)], xb, x_sem_b)
        cp.start()
        return cp

    def product_sub(xv, base, q, edge):
        conv = xv[base + q * Hq: base + (q + 1) * Hq] * k_ref[TAPS - 1]
        for t in range(TAPS - 1):
            m = TAPS - 1 - t
            start = base + q * Hq - m
            if edge and start < 0:
                sl = jnp.concatenate(
                    [jnp.zeros((m, C), f32), xv[: Hq - m]], axis=0)
            else:
                sl = xv[start: start + Hq]
            conv += sl * k_ref[t]
        return jnp.dot(
            _silu(conv).astype(bf16), w_bf, preferred_element_type=f32)

    sends = {}

    def rdma(dir_, h, q):
        comm = comm_cw if dir_ == CW else comm_ccw
        snd = send_cw if dir_ == CW else send_ccw
        rcv = recv_cw if dir_ == CW else recv_ccw
        tgt = right if dir_ == CW else left
        return pltpu.make_async_remote_copy(
            src_ref=comm.at[h % 2, R(q)],
            dst_ref=comm.at[(h + 1) % 2, R(q)],
            send_sem=snd.at[h, q],
            recv_sem=rcv.at[h, q],
            device_id=(tgt,),
            device_id_type=pl.DeviceIdType.MESH,
        )

    def start_send(dir_, h, q):
        desc = rdma(dir_, h, q)
        desc.start()
        sends[(dir_, h, q)] = desc

    barrier = pltpu.get_barrier_semaphore()
    for nbr in (left, right):
        pl.semaphore_signal(
            barrier, inc=1, device_id=(nbr,),
            device_id_type=pl.DeviceIdType.MESH,
        )
    pl.semaphore_wait(barrier, 2)

    cpT = load_top(d)
    cpB = load_bot(d)
    cpT.wait()
    xvT = xs[...]
    comm_cw[0, R(0)] = product_sub(xvT, 0, 0, True).astype(bf16)
    start_send(CW, 0, 0)
    cpB.wait()
    xvB = xb[...]
    comm_ccw[0, R(0)] = product_sub(xvB, HALO, 0, False).astype(bf16)
    start_send(CCW, 0, 0)
    for q in range(1, Q):
        comm_cw[0, R(q)] = product_sub(xvT, 0, q, True).astype(bf16)
        start_send(CW, 0, q)
        comm_ccw[0, R(q)] = product_sub(xvB, HALO, q, False).astype(bf16)
        start_send(CCW, 0, q)

    for h in range(N_DEV - 1):
        rs_ = (h + 1) % 2
        cpT = load_top((d - h - 1) % N_DEV)
        cpB = load_bot((d + h + 1) % N_DEV)
        cpT.wait()
        xvT = xs[...]
        cpB.wait()
        xvB = xb[...]
        for q in range(Q):
            addT = product_sub(xvT, 0, q, True)
            rdma(CW, h, q).wait_recv()
            if h >= 1:
                sends[(CW, h - 1, q)].wait_send()
            comm_cw[rs_, R(q)] = (
                comm_cw[rs_, R(q)].astype(f32) + addT).astype(bf16)
            start_send(CW, h + 1, q)
            addB = product_sub(xvB, HALO, q, False)
            rdma(CCW, h, q).wait_recv()
            if h >= 1:
                sends[(CCW, h - 1, q)].wait_send()
            comm_ccw[rs_, R(q)] = (
                comm_ccw[rs_, R(q)].astype(f32) + addB).astype(bf16)
            start_send(CCW, h + 1, q)

    stores_cw = []
    stores_ccw = []

    def do_store(pending, s):
        slot, c_cw, c_ccw = pending
        if s >= 2:
            stores_cw[s - 2].wait()
            stores_ccw[s - 2].wait()
        stage_cw[s % 2] = comm_cw[slot].astype(f32)
        st = pltpu.make_async_copy(
            stage_cw.at[s % 2], o_ref.at[c_cw, pl.ds(0, H)], st_cw.at[s])
        st.start()
        stores_cw.append(st)
        stage_ccw[s % 2] = comm_ccw[slot].astype(f32)
        st = pltpu.make_async_copy(
            stage_ccw.at[s % 2], o_ref.at[c_ccw, pl.ds(H, H)], st_ccw.at[s])
        st.start()
        stores_ccw.append(st)

    pending = (1, (d + 1) % N_DEV, (d - 1) % N_DEV)
    for h in (N_DEV - 1, N_DEV):
        do_store(pending, s=h - (N_DEV - 1))
        for q in range(Q):
            rdma(CW, h, q).wait_recv()
            start_send(CW, h + 1, q)
            rdma(CCW, h, q).wait_recv()
            start_send(CCW, h + 1, q)
        j = h - (N_DEV - 1)
        pending = ((h + 1) % 2, (d - j) % N_DEV, (d + j) % N_DEV)

    h = N_DEV + 1
    do_store(pending, s=2)
    for q in range(Q):
        rdma(CW, h, q).wait_recv()
        rdma(CCW, h, q).wait_recv()
    do_store((0, (d - 2) % N_DEV, (d + 2) % N_DEV), s=3)

    for dir_ in (CW, CCW):
        for hh in range(2, 2 * (N_DEV - 1)):
            for q in range(Q):
                sends[(dir_, hh, q)].wait_send()
    for st in stores_cw[-2:] + stores_ccw[-2:]:
        st.wait()


def kernel(x, k, Wp):
    B, S, C = x.shape
    Co = Wp.shape[1]
    H = S // 2
    n_hops = 2 * (N_DEV - 1)
    return pl.pallas_call(
        _fused_body,
        in_specs=[
            pl.BlockSpec(memory_space=pl.ANY),
            pl.BlockSpec(memory_space=pltpu.VMEM),
            pl.BlockSpec(memory_space=pltpu.VMEM),
        ],
        out_specs=pl.BlockSpec(memory_space=pl.ANY),
        out_shape=jax.ShapeDtypeStruct((B, S, Co), jnp.float32),
        scratch_shapes=[
            pltpu.VMEM((2, H, Co), jnp.bfloat16),
            pltpu.VMEM((2, H, Co), jnp.bfloat16),
            pltpu.VMEM((H, C), jnp.float32),
            pltpu.VMEM((H + HALO, C), jnp.float32),
            pltpu.VMEM((2, H, Co), jnp.float32),
            pltpu.VMEM((2, H, Co), jnp.float32),
            pltpu.SemaphoreType.DMA((n_hops, Q)),
            pltpu.SemaphoreType.DMA((n_hops, Q)),
            pltpu.SemaphoreType.DMA((n_hops, Q)),
            pltpu.SemaphoreType.DMA((n_hops, Q)),
            pltpu.SemaphoreType.DMA,
            pltpu.SemaphoreType.DMA,
            pltpu.SemaphoreType.DMA((4,)),
            pltpu.SemaphoreType.DMA((4,)),
        ],
        compiler_params=pltpu.CompilerParams(
            collective_id=0, vmem_limit_bytes=110 * 1024 * 1024
        ),
    )(x, k, Wp)


# device time: 172301 ns/iter; 1.1734x vs baseline; 1.0133x over previous
import jax
import jax.numpy as jnp
from jax import lax
from jax.experimental import pallas as pl
from jax.experimental.pallas import tpu as pltpu

N_DEV = 4
TAPS = 4
HALO = 8
Q = 4
CW, CCW = 0, 1


def _silu(v):
    return v * jax.nn.sigmoid(v)


def _fused_body(
    x_ref, k_ref, w_ref, o_ref,
    comm_cw, comm_ccw, xs, xb, stage_cw, stage_ccw,
    send_cw, recv_cw, send_ccw, recv_ccw, x_sem_t, x_sem_b, st_cw, st_ccw,
):
    d = lax.axis_index("i")
    left = (d - 1) % N_DEV
    right = (d + 1) % N_DEV
    B, S, C = x_ref.shape
    Co = w_ref.shape[1]
    H = S // 2
    Hq = H // Q
    f32 = jnp.float32
    bf16 = jnp.bfloat16
    w_bf = w_ref[...].astype(bf16)

    def R(q):
        return pl.ds(q * Hq, Hq)

    def load_top(b):
        cp = pltpu.make_async_copy(x_ref.at[b, pl.ds(0, H)], xs, x_sem_t)
        cp.start()
        return cp

    def load_bot(b):
        cp = pltpu.make_async_copy(
            x_ref.at[b, pl.ds(H - HALO, H + HALO)], xb, x_sem_b)
        cp.start()
        return cp

    def product_sub(xv, base, q, edge):
        conv = xv[base + q * Hq: base + (q + 1) * Hq] * k_ref[TAPS - 1]
        for t in range(TAPS - 1):
            m = TAPS - 1 - t
            start = base + q * Hq - m
            if edge and start < 0:
                sl = jnp.concatenate(
                    [jnp.zeros((m, C), f32), xv[: Hq - m]], axis=0)
            else:
                sl = xv[start: start + Hq]
            conv += sl * k_ref[t]
        return jnp.dot(
            _silu(conv).astype(bf16), w_bf, preferred_element_type=f32)

    sends = {}

    def rdma(dir_, h, q):
        comm = comm_cw if dir_ == CW else comm_ccw
        snd = send_cw if dir_ == CW else send_ccw
        rcv = recv_cw if dir_ == CW else recv_ccw
        tgt = right if dir_ == CW else left
        return pltpu.make_async_remote_copy(
            src_ref=comm.at[h % 2, R(q)],
            dst_ref=comm.at[(h + 1) % 2, R(q)],
            send_sem=snd.at[h, q],
            recv_sem=rcv.at[h, q],
            device_id=(tgt,),
            device_id_type=pl.DeviceIdType.MESH,
        )

    def start_send(dir_, h, q):
        desc = rdma(dir_, h, q)
        desc.start()
        sends[(dir_, h, q)] = desc

    barrier = pltpu.get_barrier_semaphore()
    for nbr in (left, right):
        pl.semaphore_signal(
            barrier, inc=1, device_id=(nbr,),
            device_id_type=pl.DeviceIdType.MESH,
        )
    pl.semaphore_wait(barrier, 2)

    cpT = load_top(d)
    cpB = load_bot(d)
    cpT.wait()
    xvT = xs[...]
    comm_cw[0, R(0)] = product_sub(xvT, 0, 0, True).astype(bf16)
    start_send(CW, 0, 0)
    cpB.wait()
    xvB = xb[...]
    comm_ccw[0, R(0)] = product_sub(xvB, HALO, 0, False).astype(bf16)
    start_send(CCW, 0, 0)
    for q in range(1, Q):
        comm_cw[0, R(q)] = product_sub(xvT, 0, q, True).astype(bf16)
        start_send(CW, 0, q)
        comm_ccw[0, R(q)] = product_sub(xvB, HALO, q, False).astype(bf16)
        start_send(CCW, 0, q)

    for h in range(N_DEV - 1):
        rs_ = (h + 1) % 2
        cpT = load_top((d - h - 1) % N_DEV)
        cpB = load_bot((d + h + 1) % N_DEV)
        cpT.wait()
        xvT = xs[...]
        cpB.wait()
        xvB = xb[...]
        for q in range(Q):
            addT = product_sub(xvT, 0, q, True)
            rdma(CW, h, q).wait_recv()
            if h >= 1:
                sends[(CW, h - 1, q)].wait_send()
            comm_cw[rs_, R(q)] = (
                comm_cw[rs_, R(q)].astype(f32) + addT).astype(bf16)
            start_send(CW, h + 1, q)
            addB = product_sub(xvB, HALO, q, False)
            rdma(CCW, h, q).wait_recv()
            if h >= 1:
                sends[(CCW, h - 1, q)].wait_send()
            comm_ccw[rs_, R(q)] = (
                comm_ccw[rs_, R(q)].astype(f32) + addB).astype(bf16)
            start_send(CCW, h + 1, q)

    stores_cw = []
    stores_ccw = []

    def do_store(pending, s):
        slot, c_cw, c_ccw = pending
        if s >= 2:
            stores_cw[s - 2].wait()
            stores_ccw[s - 2].wait()
        stage_cw[s % 2] = comm_cw[slot].astype(f32)
        st = pltpu.make_async_copy(
            stage_cw.at[s % 2], o_ref.at[c_cw, pl.ds(0, H)], st_cw.at[s])
        st.start()
        stores_cw.append(st)
        stage_ccw[s % 2] = comm_ccw[slot].astype(f32)
        st = pltpu.make_async_copy(
            stage_ccw.at[s % 2], o_ref.at[c_ccw, pl.ds(H, H)], st_ccw.at[s])
        st.start()
        stores_ccw.append(st)

    pending = (1, (d + 1) % N_DEV, (d - 1) % N_DEV)
    for h in (N_DEV - 1, N_DEV):
        do_store(pending, s=h - (N_DEV - 1))
        for q in range(Q):
            rdma(CW, h, q).wait_recv()
            start_send(CW, h + 1, q)
            rdma(CCW, h, q).wait_recv()
            start_send(CCW, h + 1, q)
        j = h - (N_DEV - 1)
        pending = ((h + 1) % 2, (d - j) % N_DEV, (d + j) % N_DEV)

    h = N_DEV + 1
    do_store(pending, s=2)
    stores_cw[1].wait()
    stores_ccw[1].wait()
    c_cw = (d - 2) % N_DEV
    c_ccw = (d + 2) % N_DEV
    tail = []
    for q in range(Q):
        rdma(CW, h, q).wait_recv()
        stage_cw[1, R(q)] = comm_cw[0, R(q)].astype(f32)
        st = pltpu.make_async_copy(
            stage_cw.at[1, R(q)],
            o_ref.at[c_cw, pl.ds(q * Hq, Hq)],
            st_cw.at[3 + q],
        )
        st.start()
        tail.append(st)
        rdma(CCW, h, q).wait_recv()
        stage_ccw[1, R(q)] = comm_ccw[0, R(q)].astype(f32)
        st = pltpu.make_async_copy(
            stage_ccw.at[1, R(q)],
            o_ref.at[c_ccw, pl.ds(H + q * Hq, Hq)],
            st_ccw.at[3 + q],
        )
        st.start()
        tail.append(st)

    for dir_ in (CW, CCW):
        for hh in range(2, 2 * (N_DEV - 1)):
            for q in range(Q):
                sends[(dir_, hh, q)].wait_send()
    for st in [stores_cw[2], stores_ccw[2]] + tail:
        st.wait()


def kernel(x, k, Wp):
    B, S, C = x.shape
    Co = Wp.shape[1]
    H = S // 2
    n_hops = 2 * (N_DEV - 1)
    return pl.pallas_call(
        _fused_body,
        in_specs=[
            pl.BlockSpec(memory_space=pl.ANY),
            pl.BlockSpec(memory_space=pltpu.VMEM),
            pl.BlockSpec(memory_space=pltpu.VMEM),
        ],
        out_specs=pl.BlockSpec(memory_space=pl.ANY),
        out_shape=jax.ShapeDtypeStruct((B, S, Co), jnp.float32),
        scratch_shapes=[
            pltpu.VMEM((2, H, Co), jnp.bfloat16),
            pltpu.VMEM((2, H, Co), jnp.bfloat16),
            pltpu.VMEM((H, C), jnp.float32),
            pltpu.VMEM((H + HALO, C), jnp.float32),
            pltpu.VMEM((2, H, Co), jnp.float32),
            pltpu.VMEM((2, H, Co), jnp.float32),
            pltpu.SemaphoreType.DMA((n_hops, Q)),
            pltpu.SemaphoreType.DMA((n_hops, Q)),
            pltpu.SemaphoreType.DMA((n_hops, Q)),
            pltpu.SemaphoreType.DMA((n_hops, Q)),
            pltpu.SemaphoreType.DMA,
            pltpu.SemaphoreType.DMA,
            pltpu.SemaphoreType.DMA((3 + Q,)),
            pltpu.SemaphoreType.DMA((3 + Q,)),
        ],
        compiler_params=pltpu.CompilerParams(
            collective_id=0, vmem_limit_bytes=110 * 1024 * 1024
        ),
    )(x, k, Wp)
